# Initial kernel scaffold; baseline (speedup 1.0000x reference)
import functools

import jax
import jax.numpy as jnp
from jax import lax
from jax.experimental import pallas as pl
from jax.experimental.pallas import tpu as pltpu
from jax.experimental.pallas import tpu_sc as plsc

B = 8
H = 224
W = 224
C = 192
P = B * H * W
PIX_PER_IMG = H * W
NC = 2
NS = 16
NW = NC * NS
PIX_PER_W = P // NW
CHUNK = 64
N_CHUNKS = PIX_PER_W // CHUNK
LANES = 16
SCALE = 2.0 / (H - 1.0)
HALF = 0.5 * (H - 1.0)


def _sc_body(img_hbm, tform_hbm, out_hbm, tform_v, idx_v, w_v,
             g0, g1, g2, g3, o_v, gsem):
    wid = lax.axis_index("s") * NC + lax.axis_index("c")
    wbase = wid * PIX_PER_W
    b = wid // 4
    bbase = b * PIX_PER_IMG

    pltpu.sync_copy(tform_hbm, tform_v)
    m00 = tform_v[b, 0]
    m01 = tform_v[b, 1]
    m10 = tform_v[b, 2]
    m11 = tform_v[b, 3]
    v0 = tform_v[b, 4]
    v1 = tform_v[b, 5]

    def chunk_body(ci, carry):
        gbase = wbase + ci * CHUNK

        for t in range(CHUNK // LANES):
            g = gbase + t * LANES + lax.iota(jnp.int32, (LANES,))
            rel = g - bbase
            i_i = rel // W
            j_i = rel % W
            xt = i_i.astype(jnp.float32) * SCALE - 1.0
            yt = j_i.astype(jnp.float32) * SCALE - 1.0
            xs = m00 * xt + m01 * yt + v0
            ys = m10 * xt + m11 * yt + v1
            x = HALF * (xs + 1.0)
            y = HALF * (ys + 1.0)

            xh = x + 0.5
            x0i = xh.astype(jnp.int32)
            x0i = x0i - jnp.where(x0i.astype(jnp.float32) > xh, 1, 0)
            yh = y + 0.5
            y0i = yh.astype(jnp.int32)
            y0i = y0i - jnp.where(y0i.astype(jnp.float32) > yh, 1, 0)

            x0 = jnp.clip(x0i, 0, H)
            x1 = jnp.clip(x0i + 1, 0, H)
            y0 = jnp.clip(y0i, 0, W)
            y1 = jnp.clip(y0i + 1, 0, W)

            wx0 = x0.astype(jnp.float32)
            wx1 = x1.astype(jnp.float32)
            wy0 = y0.astype(jnp.float32)
            wy1 = y1.astype(jnp.float32)
            ax0 = wx1 - x
            ax1 = x - wx0
            ay0 = wy1 - y
            ay1 = y - wy0

            zero = jnp.zeros((LANES,), jnp.float32)
            vx0 = x0 < H
            vx1 = x1 < H
            vy0 = y0 < W
            vy1 = y1 < W
            w00 = jnp.where(vx0 & vy0, ax0 * ay0, zero)
            w01 = jnp.where(vx0 & vy1, ax0 * ay1, zero)
            w10 = jnp.where(vx1 & vy0, ax1 * ay0, zero)
            w11 = jnp.where(vx1 & vy1, ax1 * ay1, zero)

            xg0 = jnp.minimum(x0, H - 1)
            xg1 = jnp.minimum(x1, H - 1)
            yg0 = jnp.minimum(y0, W - 1)
            yg1 = jnp.minimum(y1, W - 1)
            base_b = bbase + xg0 * W
            base_b1 = bbase + xg1 * W
            sl = pl.ds(t * LANES, LANES)
            idx_v[0, sl] = base_b + yg0
            idx_v[1, sl] = base_b + yg1
            idx_v[2, sl] = base_b1 + yg0
            idx_v[3, sl] = base_b1 + yg1
            w_v[0, sl] = w00
            w_v[1, sl] = w01
            w_v[2, sl] = w10
            w_v[3, sl] = w11

        c0 = pltpu.async_copy(img_hbm.at[idx_v.at[0]], g0, gsem)
        c1 = pltpu.async_copy(img_hbm.at[idx_v.at[1]], g1, gsem)
        c2 = pltpu.async_copy(img_hbm.at[idx_v.at[2]], g2, gsem)
        c3 = pltpu.async_copy(img_hbm.at[idx_v.at[3]], g3, gsem)
        c0.wait()
        c1.wait()
        c2.wait()
        c3.wait()

        def blend(p, carry2):
            w0 = jnp.full((LANES,), w_v[0, p], jnp.float32)
            w1 = jnp.full((LANES,), w_v[1, p], jnp.float32)
            w2 = jnp.full((LANES,), w_v[2, p], jnp.float32)
            w3 = jnp.full((LANES,), w_v[3, p], jnp.float32)
            for cg in range(C // LANES):
                s = pl.ds(cg * LANES, LANES)
                o_v[p, s] = (g0[p, s] * w0 + g1[p, s] * w1
                             + g2[p, s] * w2 + g3[p, s] * w3)
            return carry2

        lax.fori_loop(0, CHUNK, blend, 0, unroll=False)

        pltpu.sync_copy(o_v, out_hbm.at[pl.ds(gbase, CHUNK)])
        return carry

    lax.fori_loop(0, N_CHUNKS, chunk_body, 0, unroll=False)


@jax.jit
def _apply2dtform_sc(img_flat, tform):
    mesh = plsc.VectorSubcoreMesh(core_axis_name="c", subcore_axis_name="s",
                                  num_cores=NC, num_subcores=NS)
    kfn = pl.kernel(
        _sc_body,
        out_type=jax.ShapeDtypeStruct((P, C), jnp.float32),
        mesh=mesh,
        scratch_types=[
            pltpu.VMEM((B, 8), jnp.float32),
            pltpu.VMEM((4, CHUNK), jnp.int32),
            pltpu.VMEM((4, CHUNK), jnp.float32),
            pltpu.VMEM((CHUNK, C), jnp.float32),
            pltpu.VMEM((CHUNK, C), jnp.float32),
            pltpu.VMEM((CHUNK, C), jnp.float32),
            pltpu.VMEM((CHUNK, C), jnp.float32),
            pltpu.VMEM((CHUNK, C), jnp.float32),
            pltpu.SemaphoreType.DMA,
        ],
    )
    return kfn(img_flat, tform)


def kernel(Img, Tform):
    img_flat = Img.reshape(P, C)
    tform_pad = jnp.pad(Tform, ((0, 0), (0, 2)))
    out = _apply2dtform_sc(img_flat, tform_pad)
    return out.reshape(B, H, W, C)


# speedup vs baseline: 1.4457x; 1.4457x over previous
import functools

import jax
import jax.numpy as jnp
from jax import lax
from jax.experimental import pallas as pl
from jax.experimental.pallas import tpu as pltpu
from jax.experimental.pallas import tpu_sc as plsc

B = 8
H = 224
W = 224
C = 192
P = B * H * W
PIX_PER_IMG = H * W
NC = 2
NS = 16
NW = NC * NS
PIX_PER_W = P // NW
CHUNK = 64
N_CHUNKS = PIX_PER_W // CHUNK
LANES = 16
SCALE = 2.0 / (H - 1.0)
HALF = 0.5 * (H - 1.0)


def _bf16_round(v):
    u = lax.bitcast_convert_type(v, jnp.int32)
    r = (u >> 16) & 1
    u = (u + 32767 + r) & jnp.int32(-65536)
    return lax.bitcast_convert_type(u, jnp.float32)


def _rne_int(x):
    big = jnp.float32(2.0 ** 23)
    pos = (x + big) - big
    neg = (x - big) + big
    return jnp.where(x >= 0.0, pos, neg).astype(jnp.int32)


def _sc_body(img_hbm, tform_hbm, out_hbm, tform_v, idx_v, w_v,
             g0, g1, g2, g3, o_v, gsem):
    wid = lax.axis_index("s") * NC + lax.axis_index("c")
    wbase = wid * PIX_PER_W
    b = wid // 4
    bbase = b * PIX_PER_IMG

    pltpu.sync_copy(tform_hbm, tform_v)
    trow = tform_v[b, :]
    trow_b = _bf16_round(trow)
    m00 = trow_b[0]
    m01 = trow_b[1]
    m10 = trow_b[2]
    m11 = trow_b[3]
    v0 = trow[4]
    v1 = trow[5]

    def chunk_body(ci, carry):
        gbase = wbase + ci * CHUNK

        for t in range(CHUNK // LANES):
            g = gbase + t * LANES + lax.iota(jnp.int32, LANES)
            rel = g - bbase
            i_i = rel // W
            j_i = rel % W
            si = i_i.astype(jnp.float32) / 223.0
            sj = j_i.astype(jnp.float32) / 223.0
            xt = _bf16_round(si - (1.0 - si))
            yt = _bf16_round(sj - (1.0 - sj))
            xs = (m00 * xt + m01 * yt) + v0
            ys = (m10 * xt + m11 * yt) + v1
            x = (0.5 * (xs + 1.0)) * 223.0
            y = (0.5 * (ys + 1.0)) * 223.0

            x0i = _rne_int(x)
            y0i = _rne_int(y)

            x0 = jnp.clip(x0i, 0, H)
            x1 = jnp.clip(x0i + 1, 0, H)
            y0 = jnp.clip(y0i, 0, W)
            y1 = jnp.clip(y0i + 1, 0, W)

            wx0 = x0.astype(jnp.float32)
            wx1 = x1.astype(jnp.float32)
            wy0 = y0.astype(jnp.float32)
            wy1 = y1.astype(jnp.float32)
            ax0 = wx1 - x
            ax1 = x - wx0
            ay0 = wy1 - y
            ay1 = y - wy0

            zero = jnp.zeros((LANES,), jnp.float32)
            vx0 = x0 < H
            vx1 = x1 < H
            vy0 = y0 < W
            vy1 = y1 < W
            w00 = jnp.where(vx0 & vy0, ax0 * ay0, zero)
            w01 = jnp.where(vx0 & vy1, ax0 * ay1, zero)
            w10 = jnp.where(vx1 & vy0, ax1 * ay0, zero)
            w11 = jnp.where(vx1 & vy1, ax1 * ay1, zero)

            xg0 = jnp.minimum(x0, H - 1)
            xg1 = jnp.minimum(x1, H - 1)
            yg0 = jnp.minimum(y0, W - 1)
            yg1 = jnp.minimum(y1, W - 1)
            base_b = bbase + xg0 * W
            base_b1 = bbase + xg1 * W
            sl = pl.ds(t * LANES, LANES)
            idx_v[0, sl] = base_b + yg0
            idx_v[1, sl] = base_b + yg1
            idx_v[2, sl] = base_b1 + yg0
            idx_v[3, sl] = base_b1 + yg1
            w_v[0, sl] = w00
            w_v[1, sl] = w01
            w_v[2, sl] = w10
            w_v[3, sl] = w11

        c0 = pltpu.async_copy(img_hbm.at[idx_v.at[0]], g0, gsem)
        c1 = pltpu.async_copy(img_hbm.at[idx_v.at[1]], g1, gsem)
        c2 = pltpu.async_copy(img_hbm.at[idx_v.at[2]], g2, gsem)
        c3 = pltpu.async_copy(img_hbm.at[idx_v.at[3]], g3, gsem)
        c0.wait()
        c1.wait()
        c2.wait()
        c3.wait()

        def blend(p, carry2):
            pidx = jnp.full((LANES,), p, jnp.int32)
            k0 = jnp.zeros((LANES,), jnp.int32)
            w0 = plsc.load_gather(w_v, [k0, pidx])
            w1 = plsc.load_gather(w_v, [k0 + 1, pidx])
            w2 = plsc.load_gather(w_v, [k0 + 2, pidx])
            w3 = plsc.load_gather(w_v, [k0 + 3, pidx])
            for cg in range(C // LANES):
                s = pl.ds(cg * LANES, LANES)
                o_v[p, s] = (g0[p, s] * w0 + g1[p, s] * w1
                             + g2[p, s] * w2 + g3[p, s] * w3)
            return carry2

        lax.fori_loop(0, CHUNK, blend, 0, unroll=False)

        pltpu.sync_copy(o_v, out_hbm.at[pl.ds(gbase, CHUNK)])
        return carry

    lax.fori_loop(0, N_CHUNKS, chunk_body, 0, unroll=False)


@jax.jit
def _apply2dtform_sc(img_flat, tform):
    mesh = plsc.VectorSubcoreMesh(core_axis_name="c", subcore_axis_name="s",
                                  num_cores=NC, num_subcores=NS)
    kfn = pl.kernel(
        _sc_body,
        out_type=jax.ShapeDtypeStruct((P, C), jnp.float32),
        mesh=mesh,
        compiler_params=pltpu.CompilerParams(use_tc_tiling_on_sc=False,
                                             needs_layout_passes=False),
        scratch_types=[
            pltpu.VMEM((B, 16), jnp.float32),
            pltpu.VMEM((4, CHUNK), jnp.int32),
            pltpu.VMEM((4, CHUNK), jnp.float32),
            pltpu.VMEM((CHUNK, C), jnp.float32),
            pltpu.VMEM((CHUNK, C), jnp.float32),
            pltpu.VMEM((CHUNK, C), jnp.float32),
            pltpu.VMEM((CHUNK, C), jnp.float32),
            pltpu.VMEM((CHUNK, C), jnp.float32),
            pltpu.SemaphoreType.DMA,
        ],
    )
    return kfn(img_flat, tform)


def kernel(Img, Tform):
    img_flat = Img.reshape(P, C)
    tform_pad = jnp.pad(Tform, ((0, 0), (0, 10)))
    out = _apply2dtform_sc(img_flat, tform_pad)
    return out.reshape(B, H, W, C)


# speedup vs baseline: 1.4758x; 1.0208x over previous
import functools

import jax
import jax.numpy as jnp
from jax import lax
from jax.experimental import pallas as pl
from jax.experimental.pallas import tpu as pltpu
from jax.experimental.pallas import tpu_sc as plsc

B = 8
H = 224
W = 224
C = 192
P = B * H * W
PIX_PER_IMG = H * W
NC = 2
NS = 16
NW = NC * NS
PIX_PER_W = P // NW
CHUNK = 64
N_CHUNKS = PIX_PER_W // CHUNK
LANES = 16
SCALE = 2.0 / (H - 1.0)
HALF = 0.5 * (H - 1.0)


def _bf16_round(v):
    u = lax.bitcast_convert_type(v, jnp.int32)
    r = (u >> 16) & 1
    u = (u + 32767 + r) & jnp.int32(-65536)
    return lax.bitcast_convert_type(u, jnp.float32)


def _rne_int(x):
    big = jnp.float32(2.0 ** 23)
    pos = (x + big) - big
    neg = (x - big) + big
    return jnp.where(x >= 0.0, pos, neg).astype(jnp.int32)


def _sc_body(img_hbm, tform_hbm, out_hbm, tform_v, idx_v, w_v,
             g0, g1, g2, g3, o_v, gsem):
    wid = lax.axis_index("s") * NC + lax.axis_index("c")
    wbase = wid * PIX_PER_W
    b = wid // 4
    bbase = b * PIX_PER_IMG

    pltpu.sync_copy(tform_hbm, tform_v)
    trow = tform_v[b, :]
    trow_b = _bf16_round(trow)
    m00 = trow_b[0]
    m01 = trow_b[1]
    m10 = trow_b[2]
    m11 = trow_b[3]
    v0 = trow[4]
    v1 = trow[5]

    def chunk_body(ci, carry):
        gbase = wbase + ci * CHUNK

        for t in range(CHUNK // LANES):
            g = gbase + t * LANES + lax.iota(jnp.int32, LANES)
            rel = g - bbase
            i_i = rel // W
            j_i = rel % W
            si = i_i.astype(jnp.float32) / 223.0
            sj = j_i.astype(jnp.float32) / 223.0
            xt = _bf16_round(si - (1.0 - si))
            yt = _bf16_round(sj - (1.0 - sj))
            xs = (m00 * xt + m01 * yt) + v0
            ys = (m10 * xt + m11 * yt) + v1
            x = (0.5 * (xs + 1.0)) * 223.0
            y = (0.5 * (ys + 1.0)) * 223.0

            x0i = _rne_int(x)
            y0i = _rne_int(y)

            x0 = jnp.clip(x0i, 0, H)
            x1 = jnp.clip(x0i + 1, 0, H)
            y0 = jnp.clip(y0i, 0, W)
            y1 = jnp.clip(y0i + 1, 0, W)

            wx0 = x0.astype(jnp.float32)
            wx1 = x1.astype(jnp.float32)
            wy0 = y0.astype(jnp.float32)
            wy1 = y1.astype(jnp.float32)
            ax0 = wx1 - x
            ax1 = x - wx0
            ay0 = wy1 - y
            ay1 = y - wy0

            zero = jnp.zeros((LANES,), jnp.float32)
            vx0 = x0 < H
            vx1 = x1 < H
            vy0 = y0 < W
            vy1 = y1 < W
            w00 = jnp.where(vx0 & vy0, ax0 * ay0, zero)
            w01 = jnp.where(vx0 & vy1, ax0 * ay1, zero)
            w10 = jnp.where(vx1 & vy0, ax1 * ay0, zero)
            w11 = jnp.where(vx1 & vy1, ax1 * ay1, zero)

            xg0 = jnp.minimum(x0, H - 1)
            xg1 = jnp.minimum(x1, H - 1)
            yg0 = jnp.minimum(y0, W - 1)
            yg1 = jnp.minimum(y1, W - 1)
            base_b = bbase + xg0 * W
            base_b1 = bbase + xg1 * W
            sl = pl.ds(t * LANES, LANES)
            idx_v[0, sl] = base_b + yg0
            idx_v[1, sl] = base_b + yg1
            idx_v[2, sl] = base_b1 + yg0
            idx_v[3, sl] = base_b1 + yg1
            w_v[0, sl] = w00
            w_v[1, sl] = w01
            w_v[2, sl] = w10
            w_v[3, sl] = w11

        c0 = pltpu.async_copy(img_hbm.at[idx_v.at[0]], g0, gsem)
        c1 = pltpu.async_copy(img_hbm.at[idx_v.at[1]], g1, gsem)
        c2 = pltpu.async_copy(img_hbm.at[idx_v.at[2]], g2, gsem)
        c3 = pltpu.async_copy(img_hbm.at[idx_v.at[3]], g3, gsem)
        c0.wait()
        c1.wait()
        c2.wait()
        c3.wait()

        def blend(p, carry2):
            pidx = jnp.full((LANES,), p, jnp.int32)
            k0 = jnp.zeros((LANES,), jnp.int32)
            w0 = plsc.load_gather(w_v, [k0, pidx])
            w1 = plsc.load_gather(w_v, [k0 + 1, pidx])
            w2 = plsc.load_gather(w_v, [k0 + 2, pidx])
            w3 = plsc.load_gather(w_v, [k0 + 3, pidx])
            for cg in range(C // LANES):
                s = pl.ds(cg * LANES, LANES)
                o_v[p, s] = (g0[p, s] * w0 + g1[p, s] * w1
                             + g2[p, s] * w2 + g3[p, s] * w3)
            return carry2

        lax.fori_loop(0, CHUNK, blend, 0, unroll=4)

        pltpu.sync_copy(o_v, out_hbm.at[pl.ds(gbase, CHUNK)])
        return carry

    lax.fori_loop(0, N_CHUNKS, chunk_body, 0, unroll=False)


@jax.jit
def _apply2dtform_sc(img_flat, tform):
    mesh = plsc.VectorSubcoreMesh(core_axis_name="c", subcore_axis_name="s",
                                  num_cores=NC, num_subcores=NS)
    kfn = pl.kernel(
        _sc_body,
        out_type=jax.ShapeDtypeStruct((P, C), jnp.float32),
        mesh=mesh,
        compiler_params=pltpu.CompilerParams(use_tc_tiling_on_sc=False,
                                             needs_layout_passes=False),
        scratch_types=[
            pltpu.VMEM((B, 16), jnp.float32),
            pltpu.VMEM((4, CHUNK), jnp.int32),
            pltpu.VMEM((4, CHUNK), jnp.float32),
            pltpu.VMEM((CHUNK, C), jnp.float32),
            pltpu.VMEM((CHUNK, C), jnp.float32),
            pltpu.VMEM((CHUNK, C), jnp.float32),
            pltpu.VMEM((CHUNK, C), jnp.float32),
            pltpu.VMEM((CHUNK, C), jnp.float32),
            pltpu.SemaphoreType.DMA,
        ],
    )
    return kfn(img_flat, tform)


def kernel(Img, Tform):
    img_flat = Img.reshape(P, C)
    tform_pad = jnp.pad(Tform, ((0, 0), (0, 10)))
    out = _apply2dtform_sc(img_flat, tform_pad)
    return out.reshape(B, H, W, C)
